# baseline (speedup 1.0000x reference)
import functools

import jax
import jax.numpy as jnp
from jax import lax
from jax.experimental import pallas as pl
from jax.experimental.pallas import tpu as pltpu
from jax.experimental.pallas import tpu_sc as plsc

N_EXPERTS = 8
T_TOKENS = 8192
D_MODEL = 2048
LN_EPS = 1e-5

_NC = 2
_NS = 16
_NW = _NC * _NS
_LANES = 16
_TPW = T_TOKENS // _NW
_LROWS = T_TOKENS * N_EXPERTS // 128


def _route_body(logits_hbm, oh_hbm, logits_v, oh_v):
    wid = lax.axis_index("s") * _NC + lax.axis_index("c")
    pltpu.sync_copy(logits_hbm.at[pl.ds(wid * _TPW, _TPW)], logits_v)
    lane = lax.iota(jnp.int32, _LANES)

    def step(j, carry):
        rows = lane + j * _LANES
        best = plsc.load_gather(logits_v, [rows, jnp.zeros((_LANES,), jnp.int32)])
        besti = jnp.zeros((_LANES,), jnp.int32)
        for e in range(1, N_EXPERTS):
            col = jnp.full((_LANES,), e, jnp.int32)
            v = plsc.load_gather(logits_v, [rows, col])
            m = v > best
            best = jnp.where(m, v, best)
            besti = jnp.where(m, col, besti)
        for e in range(N_EXPERTS):
            oh_v[e, pl.ds(j * _LANES, _LANES)] = jnp.where(
                besti == e, 1.0, 0.0).astype(jnp.float32)
        return carry

    lax.fori_loop(0, _TPW // _LANES, step, 0)
    pltpu.sync_copy(oh_v, oh_hbm.at[:, pl.ds(wid * _TPW, _TPW)])


@functools.cache
def _make_route():
    return pl.kernel(
        _route_body,
        out_type=jax.ShapeDtypeStruct((N_EXPERTS, T_TOKENS), jnp.float32),
        mesh=plsc.VectorSubcoreMesh(core_axis_name="c", subcore_axis_name="s"),
        scratch_types=[
            pltpu.VMEM((_TPW, N_EXPERTS), jnp.float32),
            pltpu.VMEM((N_EXPERTS, _TPW), jnp.float32),
        ],
        compiler_params=pltpu.CompilerParams(needs_layout_passes=False),
    )


_BLK = 1024


def _ln_body(oh_ref, hid_ref, w_ref, b_ref, out_ref):
    x = hid_ref[...]
    s1 = jnp.sum(x, axis=1, keepdims=True)
    s2 = jnp.sum(x * x, axis=1, keepdims=True)
    mean = s1 * (1.0 / D_MODEL)
    var = s2 * (1.0 / D_MODEL) - mean * mean
    r = lax.rsqrt(var + LN_EPS)
    oh = oh_ref[...]
    gamma = lax.dot_general(oh, w_ref[...], (((0,), (0,)), ((), ())),
                            preferred_element_type=jnp.float32)
    beta = lax.dot_general(oh, b_ref[...], (((0,), (0,)), ((), ())),
                           preferred_element_type=jnp.float32)
    out_ref[...] = (x - mean) * (r * gamma) + beta


def kernel(hidden_states, router_logits, ln_weight, ln_bias):
    oh = _make_route()(router_logits)
    return pl.pallas_call(
        _ln_body,
        grid=(T_TOKENS // _BLK,),
        in_specs=[
            pl.BlockSpec((N_EXPERTS, _BLK), lambda i: (0, i)),
            pl.BlockSpec((_BLK, D_MODEL), lambda i: (i, 0)),
            pl.BlockSpec((N_EXPERTS, D_MODEL), lambda i: (0, 0)),
            pl.BlockSpec((N_EXPERTS, D_MODEL), lambda i: (0, 0)),
        ],
        out_specs=pl.BlockSpec((_BLK, D_MODEL), lambda i: (i, 0)),
        out_shape=jax.ShapeDtypeStruct((T_TOKENS, D_MODEL), jnp.float32),
        compiler_params=pltpu.CompilerParams(
            dimension_semantics=("arbitrary",),
        ),
    )(oh, hidden_states, ln_weight, ln_bias)


# speedup vs baseline: 1.0069x; 1.0069x over previous
import functools

import jax
import jax.numpy as jnp
from jax import lax
from jax.experimental import pallas as pl
from jax.experimental.pallas import tpu as pltpu
from jax.experimental.pallas import tpu_sc as plsc

N_EXPERTS = 8
T_TOKENS = 8192
D_MODEL = 2048
LN_EPS = 1e-5

_NC = 2
_NS = 16
_NW = _NC * _NS
_LANES = 16
_TPW = T_TOKENS // _NW
_LROWS = T_TOKENS * N_EXPERTS // 128


def _route_body(logits_hbm, oh_hbm, logits_v, oh_v):
    wid = lax.axis_index("s") * _NC + lax.axis_index("c")
    pltpu.sync_copy(logits_hbm.at[pl.ds(wid * (_LROWS // _NW), _LROWS // _NW)],
                    logits_v)
    lane = lax.iota(jnp.int32, _LANES)

    def step(j, carry):
        row = jnp.full((_LANES,), j, jnp.int32)
        cols0 = lane * N_EXPERTS
        best = plsc.load_gather(logits_v, [row, cols0])
        besti = jnp.zeros((_LANES,), jnp.int32)
        for e in range(1, N_EXPERTS):
            v = plsc.load_gather(logits_v, [row, cols0 + e])
            m = v > best
            best = jnp.where(m, v, best)
            besti = jnp.where(m, jnp.full((_LANES,), e, jnp.int32), besti)
        for e in range(N_EXPERTS):
            oh_v[e, pl.ds(j * _LANES, _LANES)] = jnp.where(
                besti == e, 1.0, 0.0).astype(jnp.float32)
        return carry

    lax.fori_loop(0, _TPW // _LANES, step, 0)
    pltpu.sync_copy(oh_v, oh_hbm.at[:, pl.ds(wid * _TPW, _TPW)])


@functools.cache
def _make_route():
    return pl.kernel(
        _route_body,
        out_type=jax.ShapeDtypeStruct((N_EXPERTS, T_TOKENS), jnp.float32),
        mesh=plsc.VectorSubcoreMesh(core_axis_name="c", subcore_axis_name="s"),
        scratch_types=[
            pltpu.VMEM((_LROWS // _NW, 128), jnp.float32),
            pltpu.VMEM((N_EXPERTS, _TPW), jnp.float32),
        ],
        compiler_params=pltpu.CompilerParams(needs_layout_passes=False),
    )


_BLK = 1024


def _ln_body(oh_ref, hid_ref, w_ref, b_ref, out_ref):
    x = hid_ref[...]
    s1 = jnp.sum(x, axis=1, keepdims=True)
    s2 = jnp.sum(x * x, axis=1, keepdims=True)
    mean = s1 * (1.0 / D_MODEL)
    var = s2 * (1.0 / D_MODEL) - mean * mean
    r = lax.rsqrt(var + LN_EPS)
    oh = oh_ref[...]
    gamma = lax.dot_general(oh, w_ref[...], (((0,), (0,)), ((), ())),
                            preferred_element_type=jnp.float32)
    beta = lax.dot_general(oh, b_ref[...], (((0,), (0,)), ((), ())),
                           preferred_element_type=jnp.float32)
    out_ref[...] = (x - mean) * (r * gamma) + beta


def kernel(hidden_states, router_logits, ln_weight, ln_bias):
    oh = _make_route()(router_logits.reshape(_LROWS, 128))
    return pl.pallas_call(
        _ln_body,
        grid=(T_TOKENS // _BLK,),
        in_specs=[
            pl.BlockSpec((N_EXPERTS, _BLK), lambda i: (0, i)),
            pl.BlockSpec((_BLK, D_MODEL), lambda i: (i, 0)),
            pl.BlockSpec((N_EXPERTS, D_MODEL), lambda i: (0, 0)),
            pl.BlockSpec((N_EXPERTS, D_MODEL), lambda i: (0, 0)),
        ],
        out_specs=pl.BlockSpec((_BLK, D_MODEL), lambda i: (i, 0)),
        out_shape=jax.ShapeDtypeStruct((T_TOKENS, D_MODEL), jnp.float32),
        compiler_params=pltpu.CompilerParams(
            dimension_semantics=("parallel",),
        ),
    )(oh, hidden_states, ln_weight, ln_bias)


# speedup vs baseline: 1.0097x; 1.0028x over previous
import functools

import jax
import jax.numpy as jnp
from jax import lax
from jax.experimental import pallas as pl
from jax.experimental.pallas import tpu as pltpu
from jax.experimental.pallas import tpu_sc as plsc

N_EXPERTS = 8
T_TOKENS = 8192
D_MODEL = 2048
LN_EPS = 1e-5

_NC = 2
_NS = 16
_NW = _NC * _NS
_LANES = 16
_TPW = T_TOKENS // _NW
_LROWS = T_TOKENS * N_EXPERTS // 128


def _route_body(logits_hbm, oh_hbm, logits_v, oh_v):
    wid = lax.axis_index("s") * _NC + lax.axis_index("c")
    pltpu.sync_copy(logits_hbm.at[pl.ds(wid * (_LROWS // _NW), _LROWS // _NW)],
                    logits_v)
    lane = lax.iota(jnp.int32, _LANES)

    def step(j, carry):
        row = jnp.full((_LANES,), j, jnp.int32)
        cols0 = lane * N_EXPERTS
        best = plsc.load_gather(logits_v, [row, cols0])
        besti = jnp.zeros((_LANES,), jnp.int32)
        for e in range(1, N_EXPERTS):
            v = plsc.load_gather(logits_v, [row, cols0 + e])
            m = v > best
            best = jnp.where(m, v, best)
            besti = jnp.where(m, jnp.full((_LANES,), e, jnp.int32), besti)
        for e in range(N_EXPERTS):
            oh_v[e, pl.ds(j * _LANES, _LANES)] = jnp.where(
                besti == e, 1.0, 0.0).astype(jnp.float32)
        return carry

    lax.fori_loop(0, _TPW // _LANES, step, 0)
    pltpu.sync_copy(oh_v, oh_hbm.at[:, pl.ds(wid * _TPW, _TPW)])


@functools.cache
def _make_route():
    return pl.kernel(
        _route_body,
        out_type=jax.ShapeDtypeStruct((N_EXPERTS, T_TOKENS), jnp.float32),
        mesh=plsc.VectorSubcoreMesh(core_axis_name="c", subcore_axis_name="s"),
        scratch_types=[
            pltpu.VMEM((_LROWS // _NW, 128), jnp.float32),
            pltpu.VMEM((N_EXPERTS, _TPW), jnp.float32),
        ],
        compiler_params=pltpu.CompilerParams(needs_layout_passes=False),
    )


_BLK = 1024


def _ln_body(oh_ref, hid_ref, w_ref, b_ref, out_ref):
    x = hid_ref[...]
    s1 = jnp.sum(x, axis=1, keepdims=True)
    s2 = jnp.sum(x * x, axis=1, keepdims=True)
    mean = s1 * (1.0 / D_MODEL)
    var = s2 * (1.0 / D_MODEL) - mean * mean
    r = lax.rsqrt(var + LN_EPS)
    oh = oh_ref[...]
    gamma = lax.dot_general(oh, w_ref[...], (((0,), (0,)), ((), ())),
                            preferred_element_type=jnp.float32)
    beta = lax.dot_general(oh, b_ref[...], (((0,), (0,)), ((), ())),
                           preferred_element_type=jnp.float32)
    out_ref[...] = (x - mean) * (r * gamma) + beta


def kernel(hidden_states, router_logits, ln_weight, ln_bias):
    oh = _make_route()(router_logits.reshape(_LROWS, 128))
    return pl.pallas_call(
        _ln_body,
        grid=(T_TOKENS // _BLK,),
        in_specs=[
            pl.BlockSpec((N_EXPERTS, _BLK), lambda i: (0, i)),
            pl.BlockSpec((_BLK, D_MODEL), lambda i: (i, 0)),
            pl.BlockSpec((N_EXPERTS, D_MODEL), lambda i: (0, 0)),
            pl.BlockSpec((N_EXPERTS, D_MODEL), lambda i: (0, 0)),
        ],
        out_specs=pl.BlockSpec((_BLK, D_MODEL), lambda i: (i, 0)),
        out_shape=jax.ShapeDtypeStruct((T_TOKENS, D_MODEL), jnp.float32),
        compiler_params=pltpu.CompilerParams(
            dimension_semantics=("arbitrary",),
        ),
    )(oh, hidden_states, ln_weight, ln_bias)
